# Initial kernel scaffold; baseline (speedup 1.0000x reference)
import functools

import jax
import jax.numpy as jnp
from jax import lax
from jax.experimental import pallas as pl
from jax.experimental.pallas import tpu as pltpu
from jax.experimental.pallas import tpu_sc as plsc

F32 = jnp.float32
_HIGH = jax.lax.Precision.HIGHEST

_NC = 2
_NS = 16
_NW = _NC * _NS
_W = 128


def _sc_edge_agg(h, src2d, dst2d, *, n_pad, rows_per_w, with_deg):
    d = h.shape[1]
    x_rows = n_pad // _NS

    mesh = plsc.VectorSubcoreMesh(core_axis_name="c", subcore_axis_name="s")
    out_types = [jax.ShapeDtypeStruct((_NC, n_pad, d), F32)]
    if with_deg:
        out_types.append(jax.ShapeDtypeStruct((_NC, n_pad, 16), F32))

    scratch = [
        pltpu.VMEM((rows_per_w, _W), jnp.int32),
        pltpu.VMEM((rows_per_w, _W), jnp.int32),
        pltpu.VMEM((_W, d), F32),
        pltpu.VMEM((_W, 16), F32),
        pltpu.VMEM_SHARED((n_pad, d), F32),
        pltpu.VMEM_SHARED((n_pad, 16), F32),
        pltpu.SemaphoreType.DMA,
    ]

    def body(h_hbm, src_hbm, dst_hbm, *rest):
        if with_deg:
            out_hbm, deg_hbm, src_v, dst_v, bufa, bufc, acc, accd, sema = rest
        else:
            out_hbm, src_v, dst_v, bufa, bufc, acc, accd, sema = rest
        c = lax.axis_index("c")
        s = lax.axis_index("s")
        wid = s * _NC + c
        row0 = wid * rows_per_w

        zeros16 = jnp.zeros((16,), F32)
        one_col = jnp.where(lax.iota(jnp.int32, 16) == 0,
                            jnp.float32(1.0), jnp.float32(0.0))

        @pl.loop(0, _W)
        def _(r):
            bufc[r, pl.ds(0, 16)] = one_col

            @pl.loop(0, d, step=16)
            def _(cc):
                bufa[r, pl.ds(cc, 16)] = zeros16

        zbase = s * x_rows

        @pl.loop(0, x_rows, step=_W)
        def _(r):
            pltpu.sync_copy(bufa, acc.at[pl.ds(zbase + r, _W)])
            if with_deg:
                pltpu.sync_copy(bufa.at[:, pl.ds(0, 16)],
                                accd.at[pl.ds(zbase + r, _W)])

        plsc.subcore_barrier()

        pltpu.sync_copy(src_hbm.at[pl.ds(row0, rows_per_w)], src_v)
        pltpu.sync_copy(dst_hbm.at[pl.ds(row0, rows_per_w)], dst_v)

        @pl.loop(0, rows_per_w)
        def _(j):
            pltpu.async_copy(h_hbm.at[src_v.at[j]], bufa, sema).wait()
            pltpu.sync_copy(bufa, acc.at[dst_v.at[j]], add=True)
            if with_deg:
                pltpu.sync_copy(bufc, accd.at[dst_v.at[j]], add=True)

        plsc.subcore_barrier()

        @pl.loop(0, x_rows, step=_W)
        def _(r):
            pltpu.sync_copy(acc.at[pl.ds(zbase + r, _W)],
                            out_hbm.at[c, pl.ds(zbase + r, _W)])
            if with_deg:
                pltpu.sync_copy(accd.at[pl.ds(zbase + r, _W)],
                                deg_hbm.at[c, pl.ds(zbase + r, _W)])

    kern = pl.kernel(body, out_type=tuple(out_types), mesh=mesh,
                     scratch_types=scratch)
    return kern(h, src2d, dst2d)


def _embed_body(x_ref, w_ref, b_ref, o_ref):
    o_ref[...] = (
        jnp.dot(x_ref[...], w_ref[...], preferred_element_type=F32,
                precision=_HIGH)
        + b_ref[...]
    )


def _sage_first_body(parts_ref, pd_ref, h_ref, wl_ref, bl_ref, wr_ref,
                     o_ref, dego_ref):
    deg = pd_ref[0] + pd_ref[1]
    _sage_common(parts_ref, deg, h_ref, wl_ref, bl_ref, wr_ref, o_ref)
    dego_ref[...] = deg


def _sage_second_body(parts_ref, deg_ref, h_ref, wl_ref, bl_ref, wr_ref,
                      o_ref):
    _sage_common(parts_ref, deg_ref[...], h_ref, wl_ref, bl_ref, wr_ref,
                 o_ref)


def _sage_common(parts_ref, deg, h_ref, wl_ref, bl_ref, wr_ref, o_ref):
    summ = parts_ref[0] + parts_ref[1]
    mean = summ / jnp.maximum(deg[:, 0:1], 1.0)
    out = (
        jnp.dot(mean, wl_ref[...], preferred_element_type=F32, precision=_HIGH)
        + bl_ref[...]
        + jnp.dot(h_ref[...], wr_ref[...], preferred_element_type=F32,
                  precision=_HIGH)
    )
    o_ref[...] = jnp.maximum(out, 0.0)


def _readout_body(h_ref, batch_ref, wa_ref, wo_ref, bo_ref, ba_smem, o_ref):
    h = h_ref[...]
    s = jnp.sum(h * wa_ref[...], axis=1, keepdims=True) + ba_smem[0]
    m = jnp.max(s)
    e = jnp.exp(s - m)
    w = e / jnp.sum(e)
    wh = h * w
    batch = batch_ref[...]
    rows = []
    for gg in range(16):
        mask = (batch == gg).astype(F32)
        rows.append(jnp.sum(wh * mask, axis=0, keepdims=True))
    pooled = jnp.concatenate(rows, axis=0)
    o_ref[...] = (
        jnp.dot(pooled, wo_ref[...], preferred_element_type=F32,
                precision=_HIGH)
        + bo_ref[...]
    )


def _row_block(n):
    for b in (2000, 1000, 500, 8):
        if n % b == 0:
            return b
    return n


def kernel(x, edge_index, batch, W_emb, b_emb, Wl0, bl0, Wr0, Wl1, bl1, Wr1,
           Wa, ba, W_out, b_out):
    n, d_in = x.shape
    d_h = W_emb.shape[1]
    d_out = W_out.shape[1]
    e = edge_index.shape[1]
    g = 16

    ep = -(-e // (_NW * _W)) * (_NW * _W)
    n_pad = -(-(n + 1) // (_NS * _W)) * (_NS * _W)
    pad = ep - e
    src = edge_index[0]
    dst = edge_index[1]
    if pad:
        filler = jnp.arange(pad, dtype=jnp.int32)
        src = jnp.concatenate([src, filler % n])
        dst = jnp.concatenate([dst, n + (filler % (n_pad - n))])
    src2d = src.reshape(ep // _W, _W)
    dst2d = dst.reshape(ep // _W, _W)
    rows_per_w = ep // _W // _NW

    blk = _row_block(n)
    grid = (n // blk,)

    h0 = pl.pallas_call(
        _embed_body,
        grid=grid,
        in_specs=[
            pl.BlockSpec((blk, d_in), lambda i: (i, 0)),
            pl.BlockSpec((d_in, d_h), lambda i: (0, 0)),
            pl.BlockSpec((1, d_h), lambda i: (0, 0)),
        ],
        out_specs=pl.BlockSpec((blk, d_h), lambda i: (i, 0)),
        out_shape=jax.ShapeDtypeStruct((n, d_h), F32),
    )(x, W_emb, b_emb.reshape(1, d_h))

    parts0, pdeg0 = _sc_edge_agg(h0, src2d, dst2d, n_pad=n_pad,
                                 rows_per_w=rows_per_w, with_deg=True)
    h1, deg = pl.pallas_call(
        _sage_first_body,
        grid=grid,
        in_specs=[
            pl.BlockSpec((2, blk, d_h), lambda i: (0, i, 0)),
            pl.BlockSpec((2, blk, 16), lambda i: (0, i, 0)),
            pl.BlockSpec((blk, d_h), lambda i: (i, 0)),
            pl.BlockSpec((d_h, d_h), lambda i: (0, 0)),
            pl.BlockSpec((1, d_h), lambda i: (0, 0)),
            pl.BlockSpec((d_h, d_h), lambda i: (0, 0)),
        ],
        out_specs=[pl.BlockSpec((blk, d_h), lambda i: (i, 0)),
                   pl.BlockSpec((blk, 16), lambda i: (i, 0))],
        out_shape=[jax.ShapeDtypeStruct((n, d_h), F32),
                   jax.ShapeDtypeStruct((n, 16), F32)],
    )(parts0, pdeg0, h0, Wl0, bl0.reshape(1, d_h), Wr0)

    (parts1,) = _sc_edge_agg(h1, src2d, dst2d, n_pad=n_pad,
                             rows_per_w=rows_per_w, with_deg=False)
    h2 = pl.pallas_call(
        _sage_second_body,
        grid=grid,
        in_specs=[
            pl.BlockSpec((2, blk, d_h), lambda i: (0, i, 0)),
            pl.BlockSpec((blk, 16), lambda i: (i, 0)),
            pl.BlockSpec((blk, d_h), lambda i: (i, 0)),
            pl.BlockSpec((d_h, d_h), lambda i: (0, 0)),
            pl.BlockSpec((1, d_h), lambda i: (0, 0)),
            pl.BlockSpec((d_h, d_h), lambda i: (0, 0)),
        ],
        out_specs=pl.BlockSpec((blk, d_h), lambda i: (i, 0)),
        out_shape=jax.ShapeDtypeStruct((n, d_h), F32),
    )(parts1, deg, h1, Wl1, bl1.reshape(1, d_h), Wr1)

    out = pl.pallas_call(
        _readout_body,
        grid=(1,),
        in_specs=[
            pl.BlockSpec((n, d_h), lambda i: (0, 0)),
            pl.BlockSpec((n, 1), lambda i: (0, 0)),
            pl.BlockSpec((1, d_h), lambda i: (0, 0)),
            pl.BlockSpec((d_h, d_out), lambda i: (0, 0)),
            pl.BlockSpec((1, d_out), lambda i: (0, 0)),
            pl.BlockSpec(memory_space=pltpu.SMEM),
        ],
        out_specs=pl.BlockSpec((g, d_out), lambda i: (0, 0)),
        out_shape=jax.ShapeDtypeStruct((g, d_out), F32),
    )(h2, batch.reshape(n, 1), Wa.reshape(1, d_h), W_out,
      b_out.reshape(1, d_out), ba)
    return out


# speedup vs baseline: 6.2247x; 6.2247x over previous
import jax
import jax.numpy as jnp
from jax import lax
from jax.experimental import pallas as pl
from jax.experimental.pallas import tpu as pltpu
from jax.experimental.pallas import tpu_sc as plsc

F32 = jnp.float32
_HIGH = jax.lax.Precision.HIGHEST

_NC = 2
_NS = 16
_NW = _NC * _NS
_W = 128


def _sc_edge_agg(h, src2d, dst2d, *, n_pad, with_deg):
    d = h.shape[1]
    total_rows = src2d.shape[0]
    x_rows = n_pad // _NS

    mesh = plsc.VectorSubcoreMesh(core_axis_name="c", subcore_axis_name="s",
                                  num_cores=_NC, num_subcores=_NS)
    out_type = jax.ShapeDtypeStruct((_NC * n_pad, d), F32)

    chunk = 8
    scratch = [
        pltpu.VMEM((chunk, _W), jnp.int32),
        pltpu.VMEM((chunk, _W), jnp.int32),
        pltpu.VMEM((_W, d), F32),
        pltpu.VMEM_SHARED((n_pad, d), F32),
    ]

    def body(h_hbm, src_hbm, dst_hbm, out_hbm, src_v, dst_v, bufa, acc):
        c = lax.axis_index("c")
        s = lax.axis_index("s")
        zeros16 = jnp.zeros((16,), F32)
        one_col = jnp.where(lax.iota(jnp.int32, 16) == 0,
                            jnp.float32(1.0), jnp.float32(0.0))

        @pl.loop(0, _W)
        def _(r):
            @pl.loop(0, d, step=16)
            def _(cc):
                bufa[r, pl.ds(cc, 16)] = zeros16

        zbase = s * x_rows

        @pl.loop(0, x_rows, step=_W)
        def _(r):
            pltpu.sync_copy(bufa, acc.at[pl.ds(zbase + r, _W)])

        plsc.subcore_barrier()

        if with_deg:
            rows_w = total_rows // _NS
            row0 = s * rows_w

            @pl.when(c == 0)
            def _():
                @pl.loop(0, rows_w, step=chunk)
                def _(j0):
                    pltpu.sync_copy(src_hbm.at[pl.ds(row0 + j0, chunk)],
                                    src_v)
                    pltpu.sync_copy(dst_hbm.at[pl.ds(row0 + j0, chunk)],
                                    dst_v)

                    @pl.loop(0, chunk)
                    def _(j):
                        pltpu.sync_copy(h_hbm.at[src_v.at[j]], bufa)
                        pltpu.sync_copy(bufa, acc.at[dst_v.at[j]], add=True)

            @pl.when(c == 1)
            def _():
                @pl.loop(0, _W)
                def _(r):
                    bufa[r, pl.ds(0, 16)] = one_col

                @pl.loop(0, rows_w, step=chunk)
                def _(j0):
                    pltpu.sync_copy(dst_hbm.at[pl.ds(row0 + j0, chunk)],
                                    dst_v)

                    @pl.loop(0, chunk)
                    def _(j):
                        pltpu.sync_copy(bufa, acc.at[dst_v.at[j]], add=True)
        else:
            rows_w = total_rows // _NW
            row0 = (s * _NC + c) * rows_w

            @pl.loop(0, rows_w, step=chunk)
            def _(j0):
                pltpu.sync_copy(src_hbm.at[pl.ds(row0 + j0, chunk)], src_v)
                pltpu.sync_copy(dst_hbm.at[pl.ds(row0 + j0, chunk)], dst_v)

                @pl.loop(0, chunk)
                def _(j):
                    pltpu.sync_copy(h_hbm.at[src_v.at[j]], bufa)
                    pltpu.sync_copy(bufa, acc.at[dst_v.at[j]], add=True)

        plsc.subcore_barrier()

        @pl.loop(0, x_rows, step=_W)
        def _(r):
            pltpu.sync_copy(acc.at[pl.ds(zbase + r, _W)],
                            out_hbm.at[pl.ds(c * n_pad + zbase + r, _W)])

    kern = pl.kernel(body, out_type=out_type, mesh=mesh,
                     scratch_types=scratch)
    return kern(h, src2d, dst2d)


def _embed_body(x_ref, w_ref, b_ref, o_ref):
    o_ref[...] = (
        jnp.dot(x_ref[...], w_ref[...], preferred_element_type=F32,
                precision=_HIGH)
        + b_ref[...]
    )


def _sage_first_body(parts_ref, h_ref, wl_ref, bl_ref, wr_ref,
                     o_ref, dego_ref):
    deg = parts_ref[1, :, 0:16]
    _sage_common(parts_ref[0], deg, h_ref[...], wl_ref, bl_ref, wr_ref,
                 o_ref)
    dego_ref[...] = deg


def _sage_second_body(parts_ref, deg_ref, h_ref, wl_ref, bl_ref, wr_ref,
                      o_ref):
    summ = parts_ref[0] + parts_ref[1]
    _sage_common(summ, deg_ref[...], h_ref[...], wl_ref, bl_ref, wr_ref,
                 o_ref)


def _sage_common(summ, deg, h, wl_ref, bl_ref, wr_ref, o_ref):
    mean = summ / jnp.maximum(deg[:, 0:1], 1.0)
    out = (
        jnp.dot(mean, wl_ref[...], preferred_element_type=F32, precision=_HIGH)
        + bl_ref[...]
        + jnp.dot(h, wr_ref[...], preferred_element_type=F32,
                  precision=_HIGH)
    )
    o_ref[...] = jnp.maximum(out, 0.0)


def _readout_body(h_ref, batch_ref, wa_ref, wo_ref, bo_ref, ba_smem, o_ref):
    h = h_ref[...]
    s = jnp.sum(h * wa_ref[...], axis=1, keepdims=True) + ba_smem[0]
    m = jnp.max(s)
    e = jnp.exp(s - m)
    w = e / jnp.sum(e)
    wh = h * w
    batch = batch_ref[...]
    rows = []
    for gg in range(16):
        mask = (batch == gg).astype(F32)
        rows.append(jnp.sum(wh * mask, axis=0, keepdims=True))
    pooled = jnp.concatenate(rows, axis=0)
    o_ref[...] = (
        jnp.dot(pooled, wo_ref[...], preferred_element_type=F32,
                precision=_HIGH)
        + bo_ref[...]
    )


def _row_block(n):
    for b in (2000, 1000, 500, 8):
        if n % b == 0:
            return b
    return n


def kernel(x, edge_index, batch, W_emb, b_emb, Wl0, bl0, Wr0, Wl1, bl1, Wr1,
           Wa, ba, W_out, b_out):
    n, d_in = x.shape
    d_h = W_emb.shape[1]
    d_out = W_out.shape[1]
    e = edge_index.shape[1]
    g = 16

    ep = -(-e // (_NW * _W * 8)) * (_NW * _W * 8)
    n_pad = -(-(n + 1) // (_NS * _W)) * (_NS * _W)
    pad = ep - e
    src = edge_index[0]
    dst = edge_index[1]
    if pad:
        filler = jnp.arange(pad, dtype=jnp.int32)
        src = jnp.concatenate([src, filler % n])
        dst = jnp.concatenate([dst, n + (filler % (n_pad - n))])
    src2d = src.reshape(ep // _W, _W)
    dst2d = dst.reshape(ep // _W, _W)

    blk = _row_block(n)
    grid = (n // blk,)

    h0 = pl.pallas_call(
        _embed_body,
        grid=grid,
        in_specs=[
            pl.BlockSpec((blk, d_in), lambda i: (i, 0)),
            pl.BlockSpec((d_in, d_h), lambda i: (0, 0)),
            pl.BlockSpec((1, d_h), lambda i: (0, 0)),
        ],
        out_specs=pl.BlockSpec((blk, d_h), lambda i: (i, 0)),
        out_shape=jax.ShapeDtypeStruct((n, d_h), F32),
    )(x, W_emb, b_emb.reshape(1, d_h))

    parts0 = _sc_edge_agg(h0, src2d, dst2d, n_pad=n_pad, with_deg=True)
    parts0 = parts0.reshape(_NC, n_pad, d_h)
    h1, deg = pl.pallas_call(
        _sage_first_body,
        grid=grid,
        in_specs=[
            pl.BlockSpec((2, blk, d_h), lambda i: (0, i, 0)),
            pl.BlockSpec((blk, d_h), lambda i: (i, 0)),
            pl.BlockSpec((d_h, d_h), lambda i: (0, 0)),
            pl.BlockSpec((1, d_h), lambda i: (0, 0)),
            pl.BlockSpec((d_h, d_h), lambda i: (0, 0)),
        ],
        out_specs=[pl.BlockSpec((blk, d_h), lambda i: (i, 0)),
                   pl.BlockSpec((blk, 16), lambda i: (i, 0))],
        out_shape=[jax.ShapeDtypeStruct((n, d_h), F32),
                   jax.ShapeDtypeStruct((n, 16), F32)],
    )(parts0, h0, Wl0, bl0.reshape(1, d_h), Wr0)

    parts1 = _sc_edge_agg(h1, src2d, dst2d, n_pad=n_pad, with_deg=False)
    parts1 = parts1.reshape(_NC, n_pad, d_h)
    h2 = pl.pallas_call(
        _sage_second_body,
        grid=grid,
        in_specs=[
            pl.BlockSpec((2, blk, d_h), lambda i: (0, i, 0)),
            pl.BlockSpec((blk, 16), lambda i: (i, 0)),
            pl.BlockSpec((blk, d_h), lambda i: (i, 0)),
            pl.BlockSpec((d_h, d_h), lambda i: (0, 0)),
            pl.BlockSpec((1, d_h), lambda i: (0, 0)),
            pl.BlockSpec((d_h, d_h), lambda i: (0, 0)),
        ],
        out_specs=pl.BlockSpec((blk, d_h), lambda i: (i, 0)),
        out_shape=jax.ShapeDtypeStruct((n, d_h), F32),
    )(parts1, deg, h1, Wl1, bl1.reshape(1, d_h), Wr1)

    out = pl.pallas_call(
        _readout_body,
        grid=(1,),
        in_specs=[
            pl.BlockSpec((n, d_h), lambda i: (0, 0)),
            pl.BlockSpec((n, 1), lambda i: (0, 0)),
            pl.BlockSpec((1, d_h), lambda i: (0, 0)),
            pl.BlockSpec((d_h, d_out), lambda i: (0, 0)),
            pl.BlockSpec((1, d_out), lambda i: (0, 0)),
            pl.BlockSpec(memory_space=pltpu.SMEM),
        ],
        out_specs=pl.BlockSpec((g, d_out), lambda i: (0, 0)),
        out_shape=jax.ShapeDtypeStruct((g, d_out), F32),
    )(h2, batch.reshape(n, 1), Wa.reshape(1, d_h), W_out,
      b_out.reshape(1, d_out), ba)
    return out


# speedup vs baseline: 8.5369x; 1.3715x over previous
import jax
import jax.numpy as jnp
from jax import lax
from jax.experimental import pallas as pl
from jax.experimental.pallas import tpu as pltpu
from jax.experimental.pallas import tpu_sc as plsc

F32 = jnp.float32
_HIGH = jax.lax.Precision.HIGHEST

_NC = 2
_NS = 16
_NW = _NC * _NS
_W = 128


def _sc_edge_agg(h, src2d, dst2d, *, n_pad, with_deg):
    d = h.shape[1]
    total_rows = src2d.shape[0]
    x_rows = n_pad // _NS

    mesh = plsc.VectorSubcoreMesh(core_axis_name="c", subcore_axis_name="s",
                                  num_cores=_NC, num_subcores=_NS)
    out_type = jax.ShapeDtypeStruct((_NC * n_pad, d), F32)

    chunk = 8
    scratch = [
        pltpu.VMEM((chunk, _W), jnp.int32),
        pltpu.VMEM((chunk, _W), jnp.int32),
        pltpu.VMEM((_W, d), F32),
        pltpu.VMEM((_W, d), F32),
        pltpu.VMEM_SHARED((n_pad, d), F32),
        pltpu.SemaphoreType.DMA,
        pltpu.SemaphoreType.DMA,
    ]

    def body(h_hbm, src_hbm, dst_hbm, out_hbm, src_v, dst_v, bufa, bufb,
             acc, sema, semb):
        c = lax.axis_index("c")
        s = lax.axis_index("s")
        zeros16 = jnp.zeros((16,), F32)
        one_col = jnp.where(lax.iota(jnp.int32, 16) == 0,
                            jnp.float32(1.0), jnp.float32(0.0))

        @pl.loop(0, _W)
        def _(r):
            @pl.loop(0, d, step=16)
            def _(cc):
                bufa[r, pl.ds(cc, 16)] = zeros16

        zbase = s * x_rows

        @pl.loop(0, x_rows, step=_W)
        def _(r):
            pltpu.sync_copy(bufa, acc.at[pl.ds(zbase + r, _W)])

        plsc.subcore_barrier()

        if with_deg:
            rows_w = total_rows // _NS
            row0 = s * rows_w

            @pl.when(c == 0)
            def _():
                @pl.loop(0, rows_w, step=chunk)
                def _(j0):
                    pltpu.sync_copy(src_hbm.at[pl.ds(row0 + j0, chunk)],
                                    src_v)
                    pltpu.sync_copy(dst_hbm.at[pl.ds(row0 + j0, chunk)],
                                    dst_v)
                    cpa = pltpu.async_copy(h_hbm.at[src_v.at[0]], bufa, sema)

                    @pl.loop(0, chunk, step=2)
                    def _(j):
                        cpb = pltpu.async_copy(h_hbm.at[src_v.at[j + 1]],
                                               bufb, semb)
                        pltpu.make_async_copy(h_hbm.at[src_v.at[j]], bufa,
                                              sema).wait()
                        pltpu.sync_copy(bufa, acc.at[dst_v.at[j]], add=True)

                        @pl.when(j + 2 < chunk)
                        def _():
                            pltpu.async_copy(h_hbm.at[src_v.at[j + 2]],
                                             bufa, sema)

                        pltpu.make_async_copy(h_hbm.at[src_v.at[j + 1]],
                                              bufb, semb).wait()
                        pltpu.sync_copy(bufb, acc.at[dst_v.at[j + 1]],
                                        add=True)

            @pl.when(c == 1)
            def _():
                @pl.loop(0, _W)
                def _(r):
                    bufa[r, pl.ds(0, 16)] = one_col

                @pl.loop(0, rows_w, step=chunk)
                def _(j0):
                    pltpu.sync_copy(dst_hbm.at[pl.ds(row0 + j0, chunk)],
                                    dst_v)

                    @pl.loop(0, chunk)
                    def _(j):
                        pltpu.sync_copy(bufa, acc.at[dst_v.at[j]], add=True)
        else:
            rows_w = total_rows // _NW
            row0 = (s * _NC + c) * rows_w

            @pl.loop(0, rows_w, step=chunk)
            def _(j0):
                pltpu.sync_copy(src_hbm.at[pl.ds(row0 + j0, chunk)], src_v)
                pltpu.sync_copy(dst_hbm.at[pl.ds(row0 + j0, chunk)], dst_v)
                pltpu.async_copy(h_hbm.at[src_v.at[0]], bufa, sema)

                @pl.loop(0, chunk, step=2)
                def _(j):
                    pltpu.async_copy(h_hbm.at[src_v.at[j + 1]], bufb, semb)
                    pltpu.make_async_copy(h_hbm.at[src_v.at[j]], bufa,
                                          sema).wait()
                    pltpu.sync_copy(bufa, acc.at[dst_v.at[j]], add=True)

                    @pl.when(j + 2 < chunk)
                    def _():
                        pltpu.async_copy(h_hbm.at[src_v.at[j + 2]], bufa,
                                         sema)

                    pltpu.make_async_copy(h_hbm.at[src_v.at[j + 1]], bufb,
                                          semb).wait()
                    pltpu.sync_copy(bufb, acc.at[dst_v.at[j + 1]], add=True)

        plsc.subcore_barrier()

        @pl.loop(0, x_rows, step=_W)
        def _(r):
            pltpu.sync_copy(acc.at[pl.ds(zbase + r, _W)],
                            out_hbm.at[pl.ds(c * n_pad + zbase + r, _W)])

    kern = pl.kernel(body, out_type=out_type, mesh=mesh,
                     scratch_types=scratch)
    return kern(h, src2d, dst2d)


def _embed_body(x_ref, w_ref, b_ref, o_ref):
    o_ref[...] = (
        jnp.dot(x_ref[...], w_ref[...], preferred_element_type=F32,
                precision=_HIGH)
        + b_ref[...]
    )


def _sage_first_body(parts_ref, h_ref, wl_ref, bl_ref, wr_ref,
                     o_ref, dego_ref):
    deg = parts_ref[1, :, 0:16]
    _sage_common(parts_ref[0], deg, h_ref[...], wl_ref, bl_ref, wr_ref,
                 o_ref)
    dego_ref[...] = deg


def _sage_second_body(parts_ref, deg_ref, h_ref, wl_ref, bl_ref, wr_ref,
                      o_ref):
    summ = parts_ref[0] + parts_ref[1]
    _sage_common(summ, deg_ref[...], h_ref[...], wl_ref, bl_ref, wr_ref,
                 o_ref)


def _sage_common(summ, deg, h, wl_ref, bl_ref, wr_ref, o_ref):
    mean = summ / jnp.maximum(deg[:, 0:1], 1.0)
    out = (
        jnp.dot(mean, wl_ref[...], preferred_element_type=F32, precision=_HIGH)
        + bl_ref[...]
        + jnp.dot(h, wr_ref[...], preferred_element_type=F32,
                  precision=_HIGH)
    )
    o_ref[...] = jnp.maximum(out, 0.0)


def _readout_body(h_ref, batch_ref, wa_ref, wo_ref, bo_ref, ba_smem, o_ref):
    h = h_ref[...]
    s = jnp.sum(h * wa_ref[...], axis=1, keepdims=True) + ba_smem[0]
    m = jnp.max(s)
    e = jnp.exp(s - m)
    w = e / jnp.sum(e)
    wh = h * w
    batch = batch_ref[...]
    rows = []
    for gg in range(16):
        mask = (batch == gg).astype(F32)
        rows.append(jnp.sum(wh * mask, axis=0, keepdims=True))
    pooled = jnp.concatenate(rows, axis=0)
    o_ref[...] = (
        jnp.dot(pooled, wo_ref[...], preferred_element_type=F32,
                precision=_HIGH)
        + bo_ref[...]
    )


def _row_block(n):
    for b in (2000, 1000, 500, 8):
        if n % b == 0:
            return b
    return n


def kernel(x, edge_index, batch, W_emb, b_emb, Wl0, bl0, Wr0, Wl1, bl1, Wr1,
           Wa, ba, W_out, b_out):
    n, d_in = x.shape
    d_h = W_emb.shape[1]
    d_out = W_out.shape[1]
    e = edge_index.shape[1]
    g = 16

    ep = -(-e // (_NW * _W * 8)) * (_NW * _W * 8)
    n_pad = -(-(n + 1) // (_NS * _W)) * (_NS * _W)
    pad = ep - e
    src = edge_index[0]
    dst = edge_index[1]
    if pad:
        filler = jnp.arange(pad, dtype=jnp.int32)
        src = jnp.concatenate([src, filler % n])
        dst = jnp.concatenate([dst, n + (filler % (n_pad - n))])
    src2d = src.reshape(ep // _W, _W)
    dst2d = dst.reshape(ep // _W, _W)

    blk = _row_block(n)
    grid = (n // blk,)

    h0 = pl.pallas_call(
        _embed_body,
        grid=grid,
        in_specs=[
            pl.BlockSpec((blk, d_in), lambda i: (i, 0)),
            pl.BlockSpec((d_in, d_h), lambda i: (0, 0)),
            pl.BlockSpec((1, d_h), lambda i: (0, 0)),
        ],
        out_specs=pl.BlockSpec((blk, d_h), lambda i: (i, 0)),
        out_shape=jax.ShapeDtypeStruct((n, d_h), F32),
    )(x, W_emb, b_emb.reshape(1, d_h))

    parts0 = _sc_edge_agg(h0, src2d, dst2d, n_pad=n_pad, with_deg=True)
    parts0 = parts0.reshape(_NC, n_pad, d_h)
    h1, deg = pl.pallas_call(
        _sage_first_body,
        grid=grid,
        in_specs=[
            pl.BlockSpec((2, blk, d_h), lambda i: (0, i, 0)),
            pl.BlockSpec((blk, d_h), lambda i: (i, 0)),
            pl.BlockSpec((d_h, d_h), lambda i: (0, 0)),
            pl.BlockSpec((1, d_h), lambda i: (0, 0)),
            pl.BlockSpec((d_h, d_h), lambda i: (0, 0)),
        ],
        out_specs=[pl.BlockSpec((blk, d_h), lambda i: (i, 0)),
                   pl.BlockSpec((blk, 16), lambda i: (i, 0))],
        out_shape=[jax.ShapeDtypeStruct((n, d_h), F32),
                   jax.ShapeDtypeStruct((n, 16), F32)],
    )(parts0, h0, Wl0, bl0.reshape(1, d_h), Wr0)

    parts1 = _sc_edge_agg(h1, src2d, dst2d, n_pad=n_pad, with_deg=False)
    parts1 = parts1.reshape(_NC, n_pad, d_h)
    h2 = pl.pallas_call(
        _sage_second_body,
        grid=grid,
        in_specs=[
            pl.BlockSpec((2, blk, d_h), lambda i: (0, i, 0)),
            pl.BlockSpec((blk, 16), lambda i: (i, 0)),
            pl.BlockSpec((blk, d_h), lambda i: (i, 0)),
            pl.BlockSpec((d_h, d_h), lambda i: (0, 0)),
            pl.BlockSpec((1, d_h), lambda i: (0, 0)),
            pl.BlockSpec((d_h, d_h), lambda i: (0, 0)),
        ],
        out_specs=pl.BlockSpec((blk, d_h), lambda i: (i, 0)),
        out_shape=jax.ShapeDtypeStruct((n, d_h), F32),
    )(parts1, deg, h1, Wl1, bl1.reshape(1, d_h), Wr1)

    out = pl.pallas_call(
        _readout_body,
        grid=(1,),
        in_specs=[
            pl.BlockSpec((n, d_h), lambda i: (0, 0)),
            pl.BlockSpec((n, 1), lambda i: (0, 0)),
            pl.BlockSpec((1, d_h), lambda i: (0, 0)),
            pl.BlockSpec((d_h, d_out), lambda i: (0, 0)),
            pl.BlockSpec((1, d_out), lambda i: (0, 0)),
            pl.BlockSpec(memory_space=pltpu.SMEM),
        ],
        out_specs=pl.BlockSpec((g, d_out), lambda i: (0, 0)),
        out_shape=jax.ShapeDtypeStruct((g, d_out), F32),
    )(h2, batch.reshape(n, 1), Wa.reshape(1, d_h), W_out,
      b_out.reshape(1, d_out), ba)
    return out
